# baseline (speedup 1.0000x reference)
import functools

import jax
import jax.numpy as jnp
from jax import lax
from jax.experimental import pallas as pl
from jax.experimental.pallas import tpu as pltpu
from jax.experimental.pallas import tpu_sc as plsc

NC = 2
NS = 16
NW = NC * NS
L = 16
CB = 128
NB = 8

BN = 256


def _deg_body(npad, ept0, unr, edge_hbm, deg_hbm, dst_v, deg_v, sem):
  cid = lax.axis_index("c")
  sid = lax.axis_index("s")
  wid = cid * NS + sid
  pltpu.async_copy(edge_hbm.at[1, pl.ds(wid * ept0, ept0)], dst_v, sem).wait()

  zv = jnp.zeros((L,), jnp.float32)

  def zero(i, carry):
    for u in range(8):
      deg_v[pl.ds((i * 8 + u) * L, L)] = zv
    return carry

  lax.fori_loop(0, npad // (8 * L), zero, 0)

  ones = jnp.ones((L,), jnp.float32)

  def count(i, carry):
    for u in range(unr):
      idx = dst_v[pl.ds((i * unr + u) * L, L)]
      plsc.addupdate_scatter(deg_v, [idx], ones)
    return carry

  lax.fori_loop(0, ept0 // (unr * L), count, 0)
  pltpu.async_copy(deg_v, deg_hbm.at[wid], sem).wait()


def _agg_body(npad, nch, ys_hbm, src_hbm, dst_hbm, out_hbm,
              src_v, dst_v, rows, zero_v, agg_sh, tab_sh, gsem, ssem, sem, sem2):
  cid = lax.axis_index("c")
  sid = lax.axis_index("s")
  wid = cid * NS + sid
  rpt = npad // NS

  stripe = pl.ds(sid * rpt, rpt)
  pltpu.async_copy(ys_hbm.at[stripe], tab_sh.at[stripe], sem2)
  pltpu.async_copy(src_hbm.at[wid], src_v, sem).wait()
  pltpu.async_copy(dst_hbm.at[wid], dst_v, sem).wait()

  zv = jnp.zeros((L,), jnp.float32)

  def zero(i, carry):
    for u in range(8):
      zero_v[i * 8 + u] = zv
    return carry

  lax.fori_loop(0, rpt // 8, zero, 0)
  pltpu.sync_copy(zero_v, agg_sh.at[pl.ds(sid * rpt, rpt)])
  pltpu.make_async_copy(ys_hbm.at[stripe], tab_sh.at[stripe], sem2).wait()
  plsc.subcore_barrier()

  groups = nch // NB

  for b in range(NB):
    pltpu.async_copy(tab_sh.at[src_v.at[b]], rows.at[b], gsem.at[b])

  def step(g, carry):
    j0 = g * NB
    for b in range(NB):
      pltpu.make_async_copy(tab_sh.at[src_v.at[j0 + b]],
                            rows.at[b], gsem.at[b]).wait()
      pltpu.async_copy(rows.at[b], agg_sh.at[dst_v.at[j0 + b]],
                       ssem.at[b], add=True)
    for b in range(NB):
      pltpu.make_async_copy(rows.at[b], agg_sh.at[dst_v.at[j0 + b]],
                            ssem.at[b]).wait()
      pltpu.async_copy(tab_sh.at[src_v.at[j0 + NB + b]],
                       rows.at[b], gsem.at[b])
    return carry

  lax.fori_loop(0, groups - 1, step, 0)

  j0 = (groups - 1) * NB
  for b in range(NB):
    pltpu.make_async_copy(tab_sh.at[src_v.at[j0 + b]],
                          rows.at[b], gsem.at[b]).wait()
    pltpu.async_copy(rows.at[b], agg_sh.at[dst_v.at[j0 + b]],
                     ssem.at[b], add=True)
  for b in range(NB):
    pltpu.make_async_copy(rows.at[b], agg_sh.at[dst_v.at[j0 + b]],
                          ssem.at[b]).wait()

  plsc.subcore_barrier()
  pltpu.sync_copy(agg_sh.at[pl.ds(sid * rpt, rpt)],
                  out_hbm.at[cid, pl.ds(sid * rpt, rpt)])


def _make_sc_kernels(npad, ept, nch, ept0, unr):
  mesh = plsc.VectorSubcoreMesh(core_axis_name="c", subcore_axis_name="s")
  cp = pltpu.CompilerParams(needs_layout_passes=False, use_tc_tiling_on_sc=False)
  deg_k = pl.kernel(
      functools.partial(_deg_body, npad, ept0, unr),
      out_type=jax.ShapeDtypeStruct((NW, npad), jnp.float32),
      mesh=mesh,
      scratch_types=[
          pltpu.VMEM((ept0,), jnp.int32),
          pltpu.VMEM((npad,), jnp.float32),
          pltpu.SemaphoreType.DMA,
      ],
      compiler_params=cp,
  )
  agg_k = pl.kernel(
      functools.partial(_agg_body, npad, nch),
      out_type=jax.ShapeDtypeStruct((NC, npad, L), jnp.float32),
      mesh=mesh,
      scratch_types=[
          pltpu.VMEM((nch, CB), jnp.int32),
          pltpu.VMEM((nch, CB), jnp.int32),
          pltpu.VMEM((NB, CB, L), jnp.float32),
          pltpu.VMEM((npad // NS, L), jnp.float32),
          pltpu.VMEM_SHARED((npad, L), jnp.float32),
          pltpu.VMEM_SHARED((npad, L), jnp.float32),
          pltpu.SemaphoreType.DMA((NB,)),
          pltpu.SemaphoreType.DMA((NB,)),
          pltpu.SemaphoreType.DMA,
          pltpu.SemaphoreType.DMA,
      ],
      compiler_params=cp,
  )
  return deg_k, agg_k


def _dinv_of(dp):
  return lax.rsqrt(jnp.sum(dp, axis=0) + 1.0)[:, None]


def _mm1_body(x_ref, w_ref, dp_ref, out_ref, dv_ref):
  dinv = _dinv_of(dp_ref[...])
  h = jnp.dot(x_ref[...], w_ref[...], preferred_element_type=jnp.float32)
  out_ref[...] = h * dinv
  dv_ref[...] = jnp.broadcast_to(dinv, dv_ref.shape)


def _mm2_body(p_ref, ys_ref, b1_ref, w2_ref, dv_ref, out_ref):
  dinv = dv_ref[...]
  tot = p_ref[0] + p_ref[1] + ys_ref[...]
  t = jnp.maximum(tot * dinv + b1_ref[...], 0.0)
  out_ref[...] = jnp.dot(t, w2_ref[...],
                         preferred_element_type=jnp.float32) * dinv


def _out_body(c, q_ref, ys_ref, b2_ref, dv_ref, out_ref):
  dinv = dv_ref[...]
  o = (q_ref[0] + q_ref[1] + ys_ref[...]) * dinv + b2_ref[...]
  col = lax.broadcasted_iota(jnp.int32, o.shape, 1)
  om = jnp.where(col < c, o, -1e30)
  m = jnp.max(om, axis=1, keepdims=True)
  e = jnp.exp(om - m)
  s = jnp.sum(e, axis=1, keepdims=True)
  out_ref[...] = (om - m - jnp.log(s))[:, :out_ref.shape[1]]


def kernel(x, edge_index, W1, b1, W2, b2):
  n, d = x.shape
  h = W1.shape[1]
  c = W2.shape[1]
  assert h == L
  npad = ((n + BN - 1) // BN) * BN
  while npad % (NS * L) != 0:
    npad += BN
  e = edge_index.shape[1]
  ept = ((e + NW * CB - 1) // (NW * CB)) * CB
  while (ept // CB) % NB != 0:
    ept += CB
  nch = ept // CB
  epad = NW * ept

  pad = jnp.full((epad - e,), n, dtype=jnp.int32)
  srcp = jnp.concatenate([edge_index[0].astype(jnp.int32), pad])
  dstp = jnp.concatenate([edge_index[1].astype(jnp.int32), pad])
  src3 = srcp.reshape(NW, nch, CB)
  dst3 = dstp.reshape(NW, nch, CB)

  w2p = jnp.pad(W2, ((0, 0), (0, L - c)))
  b1r = b1.reshape(1, h)
  b2r = jnp.pad(b2, (0, L - c)).reshape(1, L)

  xp = jnp.pad(x, ((0, npad - n), (0, 0)))
  assert e % (NW * L) == 0
  ept0 = e // NW
  unr = 25 if ept0 % (25 * L) == 0 else (5 if ept0 % (5 * L) == 0 else 1)
  deg_k, agg_k = _make_sc_kernels(npad, ept, nch, ept0, unr)
  bp = 2048
  assert npad % bp == 0
  grid = (npad // bp,)
  dp_spec = pl.BlockSpec((NW, bp), lambda j: (0, j))
  row_spec = pl.BlockSpec((bp, L), lambda j: (j, 0))
  part_spec = pl.BlockSpec((NC, bp, L), lambda j: (0, j, 0))

  deg3 = deg_k(edge_index.astype(jnp.int32))

  ys1 = pl.pallas_call(
      _mm1_body,
      grid=grid,
      in_specs=[pl.BlockSpec((bp, d), lambda j: (j, 0)),
                pl.BlockSpec((d, h), lambda j: (0, 0)),
                dp_spec],
      out_specs=[row_spec, row_spec],
      out_shape=[jax.ShapeDtypeStruct((npad, h), jnp.float32),
                 jax.ShapeDtypeStruct((npad, L), jnp.float32)],
  )(xp, W1, deg3)
  ys1, dv16 = ys1

  p1 = agg_k(ys1, src3, dst3)

  ys2 = pl.pallas_call(
      _mm2_body,
      grid=grid,
      in_specs=[part_spec, row_spec,
                pl.BlockSpec((1, h), lambda j: (0, 0)),
                pl.BlockSpec((h, L), lambda j: (0, 0)),
                row_spec],
      out_specs=row_spec,
      out_shape=jax.ShapeDtypeStruct((npad, L), jnp.float32),
  )(p1, ys1, b1r, w2p, dv16)

  p2 = agg_k(ys2, src3, dst3)

  out = pl.pallas_call(
      functools.partial(_out_body, c),
      grid=grid,
      in_specs=[part_spec, row_spec,
                pl.BlockSpec((1, L), lambda j: (0, 0)),
                row_spec],
      out_specs=pl.BlockSpec((bp, c), lambda j: (j, 0)),
      out_shape=jax.ShapeDtypeStruct((npad, c), jnp.float32),
  )(p2, ys2, b2r, dv16)

  return out[:n]


# speedup vs baseline: 1.0160x; 1.0160x over previous
import functools

import jax
import jax.numpy as jnp
from jax import lax
from jax.experimental import pallas as pl
from jax.experimental.pallas import tpu as pltpu
from jax.experimental.pallas import tpu_sc as plsc

NC = 2
NS = 16
NW = NC * NS
L = 16
CB = 128
NB = 8

BN = 256


def _deg_body(npad, ept0, unr, edge_hbm, deg_hbm, dst_v, deg_v, sem):
  cid = lax.axis_index("c")
  sid = lax.axis_index("s")
  wid = cid * NS + sid
  pltpu.async_copy(edge_hbm.at[1, pl.ds(wid * ept0, ept0)], dst_v, sem).wait()

  zv = jnp.zeros((L,), jnp.float32)

  def zero(i, carry):
    for u in range(8):
      deg_v[pl.ds((i * 8 + u) * L, L)] = zv
    return carry

  lax.fori_loop(0, npad // (8 * L), zero, 0)

  ones = jnp.ones((L,), jnp.float32)

  def count(i, carry):
    for u in range(unr):
      idx = dst_v[pl.ds((i * unr + u) * L, L)]
      plsc.addupdate_scatter(deg_v, [idx], ones)
    return carry

  lax.fori_loop(0, ept0 // (unr * L), count, 0)
  pltpu.async_copy(deg_v, deg_hbm.at[wid], sem).wait()


def _agg_body(npad, nch, ys_hbm, src_hbm, dst_hbm, out_hbm,
              src_v, dst_v, rows, zero_v, agg_sh, tab_sh, gsem, ssem, sem, sem2):
  cid = lax.axis_index("c")
  sid = lax.axis_index("s")
  wid = cid * NS + sid
  rpt = npad // NS

  stripe = pl.ds(sid * rpt, rpt)
  pltpu.async_copy(ys_hbm.at[stripe], tab_sh.at[stripe], sem2)
  pltpu.async_copy(src_hbm.at[wid], src_v, sem).wait()
  pltpu.async_copy(dst_hbm.at[wid], dst_v, sem).wait()

  zv = jnp.zeros((L,), jnp.float32)

  def zero(i, carry):
    for u in range(8):
      zero_v[i * 8 + u] = zv
    return carry

  lax.fori_loop(0, rpt // 8, zero, 0)
  pltpu.sync_copy(zero_v, agg_sh.at[pl.ds(sid * rpt, rpt)])
  pltpu.make_async_copy(ys_hbm.at[stripe], tab_sh.at[stripe], sem2).wait()
  plsc.subcore_barrier()

  groups = nch // NB

  for b in range(NB):
    pltpu.async_copy(tab_sh.at[src_v.at[b]], rows.at[b], gsem.at[b])

  def step(g, carry):
    j0 = g * NB
    for b in range(NB):
      pltpu.make_async_copy(tab_sh.at[src_v.at[j0 + b]],
                            rows.at[b], gsem.at[b]).wait()
      pltpu.async_copy(rows.at[b], agg_sh.at[dst_v.at[j0 + b]],
                       ssem.at[b], add=True)
    for b in range(NB):
      pltpu.make_async_copy(rows.at[b], agg_sh.at[dst_v.at[j0 + b]],
                            ssem.at[b]).wait()
      pltpu.async_copy(tab_sh.at[src_v.at[j0 + NB + b]],
                       rows.at[b], gsem.at[b])
    return carry

  lax.fori_loop(0, groups - 1, step, 0)

  j0 = (groups - 1) * NB
  for b in range(NB):
    pltpu.make_async_copy(tab_sh.at[src_v.at[j0 + b]],
                          rows.at[b], gsem.at[b]).wait()
    pltpu.async_copy(rows.at[b], agg_sh.at[dst_v.at[j0 + b]],
                     ssem.at[b], add=True)
  for b in range(NB):
    pltpu.make_async_copy(rows.at[b], agg_sh.at[dst_v.at[j0 + b]],
                          ssem.at[b]).wait()

  plsc.subcore_barrier()
  pltpu.sync_copy(agg_sh.at[pl.ds(sid * rpt, rpt)],
                  out_hbm.at[cid, pl.ds(sid * rpt, rpt)])


def _make_sc_kernels(npad, ept, nch, ept0, unr):
  mesh = plsc.VectorSubcoreMesh(core_axis_name="c", subcore_axis_name="s")
  cp = pltpu.CompilerParams(needs_layout_passes=False, use_tc_tiling_on_sc=False)
  deg_k = pl.kernel(
      functools.partial(_deg_body, npad, ept0, unr),
      out_type=jax.ShapeDtypeStruct((NW, npad), jnp.float32),
      mesh=mesh,
      scratch_types=[
          pltpu.VMEM((ept0,), jnp.int32),
          pltpu.VMEM((npad,), jnp.float32),
          pltpu.SemaphoreType.DMA,
      ],
      compiler_params=cp,
  )
  agg_k = pl.kernel(
      functools.partial(_agg_body, npad, nch),
      out_type=jax.ShapeDtypeStruct((NC, npad, L), jnp.float32),
      mesh=mesh,
      scratch_types=[
          pltpu.VMEM((nch, CB), jnp.int32),
          pltpu.VMEM((nch, CB), jnp.int32),
          pltpu.VMEM((NB, CB, L), jnp.float32),
          pltpu.VMEM((npad // NS, L), jnp.float32),
          pltpu.VMEM_SHARED((npad, L), jnp.float32),
          pltpu.VMEM_SHARED((npad, L), jnp.float32),
          pltpu.SemaphoreType.DMA((NB,)),
          pltpu.SemaphoreType.DMA((NB,)),
          pltpu.SemaphoreType.DMA,
          pltpu.SemaphoreType.DMA,
      ],
      compiler_params=cp,
  )
  return deg_k, agg_k


def _dinv_of(dp):
  return lax.rsqrt(jnp.sum(dp, axis=0) + 1.0)[:, None]


def _mm1_body(x_ref, w_ref, dp_ref, out_ref):
  dinv = _dinv_of(dp_ref[...])
  h = jnp.dot(x_ref[...], w_ref[...], preferred_element_type=jnp.float32)
  out_ref[...] = h * dinv


def _mm2_body(p_ref, ys_ref, b1_ref, w2_ref, dp_ref, out_ref):
  dinv = _dinv_of(dp_ref[...])
  tot = p_ref[0] + p_ref[1] + ys_ref[...]
  t = jnp.maximum(tot * dinv + b1_ref[...], 0.0)
  out_ref[...] = jnp.dot(t, w2_ref[...],
                         preferred_element_type=jnp.float32) * dinv


def _out_body(c, q_ref, ys_ref, b2_ref, dp_ref, out_ref):
  dinv = _dinv_of(dp_ref[...])
  o = (q_ref[0] + q_ref[1] + ys_ref[...]) * dinv + b2_ref[...]
  col = lax.broadcasted_iota(jnp.int32, o.shape, 1)
  om = jnp.where(col < c, o, -1e30)
  m = jnp.max(om, axis=1, keepdims=True)
  e = jnp.exp(om - m)
  s = jnp.sum(e, axis=1, keepdims=True)
  out_ref[...] = (om - m - jnp.log(s))[:, :out_ref.shape[1]]


def kernel(x, edge_index, W1, b1, W2, b2):
  n, d = x.shape
  h = W1.shape[1]
  c = W2.shape[1]
  assert h == L
  npad = ((n + BN - 1) // BN) * BN
  while npad % (NS * L) != 0:
    npad += BN
  e = edge_index.shape[1]
  ept = ((e + NW * CB - 1) // (NW * CB)) * CB
  while (ept // CB) % NB != 0:
    ept += CB
  nch = ept // CB
  epad = NW * ept

  pad = jnp.full((epad - e,), n, dtype=jnp.int32)
  srcp = jnp.concatenate([edge_index[0].astype(jnp.int32), pad])
  dstp = jnp.concatenate([edge_index[1].astype(jnp.int32), pad])
  src3 = srcp.reshape(NW, nch, CB)
  dst3 = dstp.reshape(NW, nch, CB)

  w2p = jnp.pad(W2, ((0, 0), (0, L - c)))
  b1r = b1.reshape(1, h)
  b2r = jnp.pad(b2, (0, L - c)).reshape(1, L)

  xp = jnp.pad(x, ((0, npad - n), (0, 0)))
  assert e % (NW * L) == 0
  ept0 = e // NW
  unr = 5 if ept0 % (5 * L) == 0 else 1
  deg_k, agg_k = _make_sc_kernels(npad, ept, nch, ept0, unr)
  bp = 2048
  assert npad % bp == 0
  grid = (npad // bp,)
  dp_spec = pl.BlockSpec((NW, bp), lambda j: (0, j))
  row_spec = pl.BlockSpec((bp, L), lambda j: (j, 0))
  part_spec = pl.BlockSpec((NC, bp, L), lambda j: (0, j, 0))

  deg3 = deg_k(edge_index.astype(jnp.int32))

  ys1 = pl.pallas_call(
      _mm1_body,
      grid=grid,
      in_specs=[pl.BlockSpec((bp, d), lambda j: (j, 0)),
                pl.BlockSpec((d, h), lambda j: (0, 0)),
                dp_spec],
      out_specs=row_spec,
      out_shape=jax.ShapeDtypeStruct((npad, h), jnp.float32),
  )(xp, W1, deg3)

  p1 = agg_k(ys1, src3, dst3)

  ys2 = pl.pallas_call(
      _mm2_body,
      grid=grid,
      in_specs=[part_spec, row_spec,
                pl.BlockSpec((1, h), lambda j: (0, 0)),
                pl.BlockSpec((h, L), lambda j: (0, 0)),
                dp_spec],
      out_specs=row_spec,
      out_shape=jax.ShapeDtypeStruct((npad, L), jnp.float32),
  )(p1, ys1, b1r, w2p, deg3)

  p2 = agg_k(ys2, src3, dst3)

  out = pl.pallas_call(
      functools.partial(_out_body, c),
      grid=grid,
      in_specs=[part_spec, row_spec,
                pl.BlockSpec((1, L), lambda j: (0, 0)),
                dp_spec],
      out_specs=pl.BlockSpec((bp, c), lambda j: (j, 0)),
      out_shape=jax.ShapeDtypeStruct((npad, c), jnp.float32),
  )(p2, ys2, b2r, deg3)

  return out[:n]
